# Initial kernel scaffold; baseline (speedup 1.0000x reference)
import jax
import jax.numpy as jnp
from jax.experimental import pallas as pl


def kernel(radial_basis_0, radial_basis_1, radial_basis_2, sh_0, sh_1, sh_2, features_0, features_1, features_2, U_0, U_1, U_2, mlp_W1_0, mlp_W1_1, mlp_W1_2, mlp_b1_0, mlp_b1_1, mlp_b1_2, mlp_W2_0, mlp_W2_1, mlp_W2_2, mlp_b2_0, mlp_b2_1, mlp_b2_2, lin_in_0, lin_in_1, lin_in_2, lin_out_0, lin_out_1, lin_out_2, rms_g_0, rms_g_1, rms_g_2, centers, neighbors, message_scaling, n_atoms):
    raise NotImplementedError("write your pallas kernel here")


# speedup vs baseline: 1.0001x; 1.0001x over previous
import math
import jax
import jax.numpy as jnp
from jax.experimental import pallas as pl

L_MAX = 2
K_MAX_L = (128, 64, 32)
WIDTHS = (64, 32, 32)
N_ATOMS = 10000
EPS = 1e-6


def kernel(radial_basis_0, radial_basis_1, radial_basis_2, sh_0, sh_1, sh_2,
           features_0, features_1, features_2, U_0, U_1, U_2,
           mlp_W1_0, mlp_W1_1, mlp_W1_2, mlp_b1_0, mlp_b1_1, mlp_b1_2,
           mlp_W2_0, mlp_W2_1, mlp_W2_2, mlp_b2_0, mlp_b2_1, mlp_b2_2,
           lin_in_0, lin_in_1, lin_in_2, lin_out_0, lin_out_1, lin_out_2,
           rms_g_0, rms_g_1, rms_g_2, centers, neighbors, message_scaling, n_atoms):
    rb = [radial_basis_0, radial_basis_1, radial_basis_2]
    sh = [sh_0, sh_1, sh_2]
    feats = [features_0, features_1, features_2]
    U = [U_0, U_1, U_2]
    W1 = [mlp_W1_0, mlp_W1_1, mlp_W1_2]
    b1 = [mlp_b1_0, mlp_b1_1, mlp_b1_2]
    W2 = [mlp_W2_0, mlp_W2_1, mlp_W2_2]
    b2 = [mlp_b2_0, mlp_b2_1, mlp_b2_2]
    lin_in = [lin_in_0, lin_in_1, lin_in_2]
    lin_out = [lin_out_0, lin_out_1, lin_out_2]
    rms_g = [rms_g_0, rms_g_1, rms_g_2]
    ms = message_scaling + jnp.asarray(n_atoms - N_ATOMS, dtype=message_scaling.dtype)

    features = []
    for l in range(3):
        f = feats[l]
        msq = jnp.mean(jnp.square(f), axis=(1, 2, 3), keepdims=True)
        f = f / jnp.sqrt(msq + EPS) * rms_g[l]
        f = jnp.einsum('nijf,fg->nijg', f, lin_in[l])
        features.append(f)

    radial = []
    for l in range(3):
        x = jax.nn.silu(rb[l] @ W1[l] + b1[l])
        x = x @ W2[l] + b2[l]
        radial.append(x)

    ve = [sh[l][:, :, None] * radial[l][:, None, :] for l in range(3)]
    split = []
    for l in range(3):
        lower = [64, 32, 0][2 - l] if False else None
    split = [
        [ve[0][:, :, 64:128]],
        [ve[0][:, :, 32:64], ve[1][:, :, 32:64]],
        [ve[0][:, :, 0:32], ve[1][:, :, 0:32], ve[2][:, :, 0:32]],
    ]
    unc = []
    for l in range(3):
        st = jnp.concatenate(split[l], axis=1)
        n_sq = (l + 1) ** 2
        E, _, W = st.shape
        u = (U[l] @ st.swapaxes(0, 1).reshape(n_sq, E * W)).reshape(n_sq, E, W).swapaxes(0, 1)
        unc.append(u.reshape(E, l + 1, l + 1, W))

    indexed = [f[neighbors] for f in features]
    combined = [jnp.einsum('eijf,ejkf->eikf', unc[l], indexed[l]) / math.sqrt(l + 1)
                for l in range(3)]
    pooled = [jnp.zeros((N_ATOMS,) + c.shape[1:], dtype=c.dtype).at[centers].add(c) * ms
              for c in combined]
    out = [jnp.einsum('nijf,fg->nijg', pooled[l], lin_out[l]) for l in range(3)]
    return tuple(feats[l] + out[l] for l in range(3))


# speedup vs baseline: 61.8333x; 61.8302x over previous
import math
import jax
import jax.numpy as jnp
from jax import lax
from jax.experimental import pallas as pl
from jax.experimental.pallas import tpu as pltpu
from jax.experimental.pallas import tpu_sc as plsc

L_MAX = 2
K_MAX_L = (128, 64, 32)
WIDTHS = (64, 32, 32)
N_ATOMS = 10000
N_EDGES = 160000
EPS = 1e-6

D = 512
COLS = (64, 128, 288)
OFFS = (0, 64, 192)

NC, NS = 2, 16
NW = NC * NS

EPW = N_EDGES // NW
GC = 128
NGC = (EPW + GC - 1) // GC

EPT = N_EDGES // NS
SCC = 80
NSC = EPT // SCC
N_POOL = 10240
RPT = N_POOL // NS
GCOLS = 128


def _node_prep_body(f0, f1, f2, g0, g1, g2, w0, w1, w2, out):
    parts = []
    for f_ref, g_ref, w_ref in ((f0, g0, w0), (f1, g1, w1), (f2, g2, w2)):
        f = f_ref[...]
        ms = jnp.mean(f * f, axis=1, keepdims=True)
        f = f * lax.rsqrt(ms + EPS) * g_ref[...]
        parts.append(jnp.dot(f, w_ref[...], preferred_element_type=jnp.float32))
    parts.append(jnp.zeros((parts[0].shape[0], D - sum(COLS)), jnp.float32))
    out[...] = jnp.concatenate(parts, axis=1)


def _node_prep(f0, f1, f2, g0, g1, g2, w0, w1, w2):
    bn = 1000
    grid = (N_ATOMS // bn,)
    rep = lambda i: (0, 0)
    blk = lambda i: (i, 0)
    return pl.pallas_call(
        _node_prep_body,
        grid=grid,
        in_specs=[
            pl.BlockSpec((bn, 64), blk),
            pl.BlockSpec((bn, 128), blk),
            pl.BlockSpec((bn, 288), blk),
            pl.BlockSpec((1, 64), rep),
            pl.BlockSpec((1, 128), rep),
            pl.BlockSpec((1, 288), rep),
            pl.BlockSpec((64, 64), rep),
            pl.BlockSpec((128, 128), rep),
            pl.BlockSpec((288, 288), rep),
        ],
        out_specs=pl.BlockSpec((bn, D), blk),
        out_shape=jax.ShapeDtypeStruct((N_ATOMS, D), jnp.float32),
    )(f0, f1, f2, g0, g1, g2, w0, w1, w2)


def _gather_body(table_hbm, nbr_hbm, out_hbm, idx_v, rows_v, sem):
    c = lax.axis_index("c")
    s = lax.axis_index("s")
    wid = s * NC + c
    base = wid * EPW
    pltpu.sync_copy(nbr_hbm.at[pl.ds(base, EPW)], idx_v)

    def body(j, carry):
        e0 = jnp.minimum(j * GC, EPW - GC)
        pltpu.async_copy(table_hbm.at[idx_v.at[pl.ds(e0, GC)]], rows_v, sem).wait()
        pltpu.sync_copy(rows_v, out_hbm.at[pl.ds(base + e0, GC)])
        return carry

    lax.fori_loop(0, NGC, body, 0)


def _sc_gather(table, neighbors):
    return pl.kernel(
        _gather_body,
        out_type=jax.ShapeDtypeStruct((N_EDGES, D), jnp.float32),
        mesh=plsc.VectorSubcoreMesh(core_axis_name="c", subcore_axis_name="s"),
        scratch_types=[
            pltpu.VMEM((EPW,), jnp.int32),
            pltpu.VMEM((GC, D), jnp.float32),
            pltpu.SemaphoreType.DMA,
        ],
    )(table, neighbors)


def _edge_body(rb0, rb1, rb2, sh0, sh1, sh2, rows,
               w10, b10, w11, b11, w12, b12,
               w20, b20, w21, b21, w22, b22,
               u0, m1, m2, out):
    radial = []
    for rb, w1_, b1_, w2_, b2_ in ((rb0, w10, b10, w20, b20),
                                   (rb1, w11, b11, w21, b21),
                                   (rb2, w12, b12, w22, b22)):
        h = jnp.dot(rb[...], w1_[...], preferred_element_type=jnp.float32) + b1_[...]
        h = h * jax.nn.sigmoid(h)
        radial.append(jnp.dot(h, w2_[...], preferred_element_type=jnp.float32) + b2_[...])
    r0, r1, r2 = radial
    s0 = sh0[...]
    s1 = sh1[...]
    s2 = sh2[...]
    rows_ = rows[...]

    st2 = jnp.concatenate(
        [s0 * r0[:, 0:32]]
        + [s1[:, m:m + 1] * r1[:, 0:32] for m in range(3)]
        + [s2[:, m:m + 1] * r2[:, 0:32] for m in range(5)], axis=1)
    st1 = jnp.concatenate(
        [s0 * r0[:, 32:64]]
        + [s1[:, m:m + 1] * r1[:, 32:64] for m in range(3)], axis=1)
    unc2 = jnp.dot(st2, m2[...], preferred_element_type=jnp.float32)
    unc1 = jnp.dot(st1, m1[...], preferred_element_type=jnp.float32)
    unc0 = u0[...] * s0 * r0[:, 64:128]

    outs = [unc0 * rows_[:, 0:64]]
    for i in range(2):
        acc = None
        for j in range(2):
            u = unc1[:, (2 * i + j) * 32:(2 * i + j + 1) * 32]
            t = jnp.concatenate([u, u], axis=1) * rows_[:, 64 + 64 * j:128 + 64 * j]
            acc = t if acc is None else acc + t
        outs.append(acc)
    for i in range(3):
        acc = None
        for j in range(3):
            u = unc2[:, (3 * i + j) * 32:(3 * i + j + 1) * 32]
            t = jnp.concatenate([u, u, u], axis=1) * rows_[:, 192 + 96 * j:288 + 96 * j]
            acc = t if acc is None else acc + t
        outs.append(acc)
    outs.append(jnp.zeros((rows_.shape[0], D - sum(COLS)), jnp.float32))
    out[...] = jnp.concatenate(outs, axis=1)


def _edge_compute(rb0, rb1, rb2, sh0, sh1, sh2, rows, weights):
    be = 1000
    grid = (N_EDGES // be,)
    rep = lambda i: (0, 0)
    blk = lambda i: (i, 0)
    w_specs = [pl.BlockSpec(w.shape, rep) for w in weights]
    return pl.pallas_call(
        _edge_body,
        grid=grid,
        in_specs=[
            pl.BlockSpec((be, 8), blk),
            pl.BlockSpec((be, 8), blk),
            pl.BlockSpec((be, 8), blk),
            pl.BlockSpec((be, 1), blk),
            pl.BlockSpec((be, 3), blk),
            pl.BlockSpec((be, 5), blk),
            pl.BlockSpec((be, D), blk),
        ] + w_specs,
        out_specs=pl.BlockSpec((be, D), blk),
        out_shape=jax.ShapeDtypeStruct((N_EDGES, D), jnp.float32),
    )(rb0, rb1, rb2, sh0, sh1, sh2, rows, *weights)


def _scatter_body(comb_hbm, ctr_hbm, zeros_hbm, out_hbm, idx_v, rows_v, acc_sh, sem):
    c = lax.axis_index("c")
    s = lax.axis_index("s")
    for g in range(4):
        @pl.when(c == g // 2)
        def _(g=g):
            col0 = (g % 2 + 2 * (g // 2)) * GCOLS

            @pl.when(s == 0)
            def _():
                pltpu.sync_copy(zeros_hbm, acc_sh)

            plsc.subcore_barrier()

            def body(j, carry):
                e0 = s * EPT + j * SCC
                pltpu.sync_copy(ctr_hbm.at[pl.ds(e0, SCC)], idx_v)
                pltpu.async_copy(
                    comb_hbm.at[pl.ds(e0, SCC), pl.ds(col0, GCOLS)], rows_v, sem
                ).wait()
                pltpu.sync_copy(rows_v, acc_sh.at[idx_v], add=True)
                return carry

            lax.fori_loop(0, NSC, body, 0)
            plsc.subcore_barrier()
            r0 = s * RPT
            pltpu.sync_copy(
                acc_sh.at[pl.ds(r0, RPT)],
                out_hbm.at[pl.ds(r0, RPT), pl.ds(col0, GCOLS)])
            plsc.subcore_barrier()


def _sc_scatter(combined, centers, zeros):
    return pl.kernel(
        _scatter_body,
        out_type=jax.ShapeDtypeStruct((N_POOL, D), jnp.float32),
        mesh=plsc.VectorSubcoreMesh(core_axis_name="c", subcore_axis_name="s"),
        scratch_types=[
            pltpu.VMEM((SCC,), jnp.int32),
            pltpu.VMEM((SCC, GCOLS), jnp.float32),
            pltpu.VMEM_SHARED((N_POOL, GCOLS), jnp.float32),
            pltpu.SemaphoreType.DMA,
        ],
    )(combined, centers, zeros)


def _final_body(pooled, f0, f1, f2, w0, w1, w2, ms, o0, o1, o2):
    p = pooled[...] * ms[...]
    o0[...] = f0[...] + jnp.dot(p[:, 0:64], w0[...], preferred_element_type=jnp.float32)
    o1[...] = f1[...] + jnp.dot(p[:, 64:192], w1[...], preferred_element_type=jnp.float32)
    o2[...] = f2[...] + jnp.dot(p[:, 192:480], w2[...], preferred_element_type=jnp.float32)


def _final(pooled, f0, f1, f2, w0, w1, w2, ms):
    bn = 1000
    grid = (N_ATOMS // bn,)
    rep = lambda i: (0, 0)
    blk = lambda i: (i, 0)
    return pl.pallas_call(
        _final_body,
        grid=grid,
        in_specs=[
            pl.BlockSpec((bn, D), blk),
            pl.BlockSpec((bn, 64), blk),
            pl.BlockSpec((bn, 128), blk),
            pl.BlockSpec((bn, 288), blk),
            pl.BlockSpec((64, 64), rep),
            pl.BlockSpec((128, 128), rep),
            pl.BlockSpec((288, 288), rep),
            pl.BlockSpec((1, 1), rep),
        ],
        out_specs=[
            pl.BlockSpec((bn, 64), blk),
            pl.BlockSpec((bn, 128), blk),
            pl.BlockSpec((bn, 288), blk),
        ],
        out_shape=[
            jax.ShapeDtypeStruct((N_ATOMS, 64), jnp.float32),
            jax.ShapeDtypeStruct((N_ATOMS, 128), jnp.float32),
            jax.ShapeDtypeStruct((N_ATOMS, 288), jnp.float32),
        ],
    )(pooled, f0, f1, f2, w0, w1, w2, ms)


def kernel(radial_basis_0, radial_basis_1, radial_basis_2,
           sh_0, sh_1, sh_2,
           features_0, features_1, features_2,
           U_0, U_1, U_2,
           mlp_W1_0, mlp_W1_1, mlp_W1_2,
           mlp_b1_0, mlp_b1_1, mlp_b1_2,
           mlp_W2_0, mlp_W2_1, mlp_W2_2,
           mlp_b2_0, mlp_b2_1, mlp_b2_2,
           lin_in_0, lin_in_1, lin_in_2,
           lin_out_0, lin_out_1, lin_out_2,
           rms_g_0, rms_g_1, rms_g_2,
           centers, neighbors, message_scaling, n_atoms):
    f32 = jnp.float32
    feats = (features_0, features_1, features_2)
    f_flat = [f.reshape(N_ATOMS, COLS[l]) for l, f in enumerate(feats)]
    g_flat = [g.reshape(1, COLS[l]) for l, g in
              enumerate((rms_g_0, rms_g_1, rms_g_2))]
    eye32 = jnp.eye(32, dtype=f32)
    lin_in_k = [lin_in_0,
                jnp.kron(jnp.eye(4, dtype=f32), lin_in_1),
                jnp.kron(jnp.eye(9, dtype=f32), lin_in_2)]
    lin_out_k = [lin_out_0,
                 jnp.kron(jnp.eye(4, dtype=f32), lin_out_1),
                 jnp.kron(jnp.eye(9, dtype=f32), lin_out_2)]
    m1 = jnp.kron(U_1.T, eye32) / math.sqrt(2.0)
    m2 = jnp.kron(U_2.T, eye32) / math.sqrt(3.0)
    u0 = U_0.reshape(1, 1)
    ms = (message_scaling
          + jnp.asarray(n_atoms - N_ATOMS, message_scaling.dtype)).reshape(1, 1)

    table = _node_prep(f_flat[0], f_flat[1], f_flat[2],
                       g_flat[0], g_flat[1], g_flat[2],
                       lin_in_k[0], lin_in_k[1], lin_in_k[2])
    rows = _sc_gather(table, neighbors)
    weights = [mlp_W1_0.reshape(8, 64), mlp_b1_0.reshape(1, 64),
               mlp_W1_1.reshape(8, 64), mlp_b1_1.reshape(1, 64),
               mlp_W1_2.reshape(8, 64), mlp_b1_2.reshape(1, 64),
               mlp_W2_0.reshape(64, 128), mlp_b2_0.reshape(1, 128),
               mlp_W2_1.reshape(64, 64), mlp_b2_1.reshape(1, 64),
               mlp_W2_2.reshape(64, 32), mlp_b2_2.reshape(1, 32),
               u0, m1, m2]
    combined = _edge_compute(radial_basis_0, radial_basis_1, radial_basis_2,
                             sh_0, sh_1, sh_2, rows, weights)
    zeros = jnp.zeros((N_POOL, GCOLS), f32)
    pooled = _sc_scatter(combined, centers, zeros)[:N_ATOMS]
    o0, o1, o2 = _final(pooled, f_flat[0], f_flat[1], f_flat[2],
                        lin_out_k[0], lin_out_k[1], lin_out_k[2], ms)
    return (o0.reshape(N_ATOMS, 1, 1, 64),
            o1.reshape(N_ATOMS, 2, 2, 32),
            o2.reshape(N_ATOMS, 3, 3, 32))


# speedup vs baseline: 68.2336x; 1.1035x over previous
import math
import jax
import jax.numpy as jnp
from jax import lax
from jax.experimental import pallas as pl
from jax.experimental.pallas import tpu as pltpu
from jax.experimental.pallas import tpu_sc as plsc

L_MAX = 2
K_MAX_L = (128, 64, 32)
WIDTHS = (64, 32, 32)
N_ATOMS = 10000
N_EDGES = 160000
EPS = 1e-6

D = 512
COLS = (64, 128, 288)
OFFS = (0, 64, 192)

NC, NS = 2, 16
NW = NC * NS

EPW = N_EDGES // NW
GC = 120
NGC = (EPW + GC - 1) // GC

EPT = N_EDGES // NS
SCC = 40
NSC = EPT // SCC
N_POOL = 10240
RPT = N_POOL // NS
GCOLS = 128


def _node_prep_body(f0, f1, f2, g0, g1, g2, w0, w1, w2, out):
    parts = []
    for f_ref, g_ref, w_ref in ((f0, g0, w0), (f1, g1, w1), (f2, g2, w2)):
        f = f_ref[...]
        ms = jnp.mean(f * f, axis=1, keepdims=True)
        f = f * lax.rsqrt(ms + EPS) * g_ref[...]
        parts.append(jnp.dot(f, w_ref[...], preferred_element_type=jnp.float32))
    parts.append(jnp.zeros((parts[0].shape[0], D - sum(COLS)), jnp.float32))
    out[...] = jnp.concatenate(parts, axis=1)


def _node_prep(f0, f1, f2, g0, g1, g2, w0, w1, w2):
    bn = 1000
    grid = (N_ATOMS // bn,)
    rep = lambda i: (0, 0)
    blk = lambda i: (i, 0)
    return pl.pallas_call(
        _node_prep_body,
        grid=grid,
        in_specs=[
            pl.BlockSpec((bn, 64), blk),
            pl.BlockSpec((bn, 128), blk),
            pl.BlockSpec((bn, 288), blk),
            pl.BlockSpec((1, 64), rep),
            pl.BlockSpec((1, 128), rep),
            pl.BlockSpec((1, 288), rep),
            pl.BlockSpec((64, 64), rep),
            pl.BlockSpec((128, 128), rep),
            pl.BlockSpec((288, 288), rep),
        ],
        out_specs=pl.BlockSpec((bn, D), blk),
        out_shape=jax.ShapeDtypeStruct((N_ATOMS, D), jnp.float32),
    )(f0, f1, f2, g0, g1, g2, w0, w1, w2)


def _gather_body(table_hbm, nbr_hbm, out_hbm, idx_v, rows_a, rows_b, sem_a, sem_b):
    c = lax.axis_index("c")
    s = lax.axis_index("s")
    wid = s * NC + c
    base = wid * EPW
    bufs = (rows_a, rows_b)
    sems = (sem_a, sem_b)
    pltpu.sync_copy(nbr_hbm.at[pl.ds(base, EPW)], idx_v)

    def start(j, k):
        e0 = jnp.minimum(j * GC, EPW - GC)
        pltpu.async_copy(table_hbm.at[idx_v.at[pl.ds(e0, GC)]], bufs[k], sems[k])

    start(0, 0)
    start(1, 1)

    def body(jj, carry):
        for k in range(2):
            j = jj * 2 + k
            e0 = jnp.minimum(j * GC, EPW - GC)
            pltpu.make_async_copy(table_hbm.at[idx_v.at[pl.ds(e0, GC)]],
                                  bufs[k], sems[k]).wait()
            pltpu.sync_copy(bufs[k], out_hbm.at[pl.ds(base + e0, GC)])
            start(j + 2, k)
        return carry

    lax.fori_loop(0, NGC // 2, body, 0)
    for k in range(2):
        pltpu.make_async_copy(table_hbm.at[idx_v.at[pl.ds(0, GC)]],
                              bufs[k], sems[k]).wait()


def _sc_gather(table, neighbors):
    return pl.kernel(
        _gather_body,
        out_type=jax.ShapeDtypeStruct((N_EDGES, D), jnp.float32),
        mesh=plsc.VectorSubcoreMesh(core_axis_name="c", subcore_axis_name="s"),
        scratch_types=[
            pltpu.VMEM((EPW,), jnp.int32),
            pltpu.VMEM((GC, D), jnp.float32),
            pltpu.VMEM((GC, D), jnp.float32),
            pltpu.SemaphoreType.DMA,
            pltpu.SemaphoreType.DMA,
        ],
    )(table, neighbors)


def _edge_body(rb0, rb1, rb2, sh0, sh1, sh2, rows,
               w10, b10, w11, b11, w12, b12,
               w20, b20, w21, b21, w22, b22,
               u0, m1, m2, out):
    radial = []
    for rb, w1_, b1_, w2_, b2_ in ((rb0, w10, b10, w20, b20),
                                   (rb1, w11, b11, w21, b21),
                                   (rb2, w12, b12, w22, b22)):
        h = jnp.dot(rb[...], w1_[...], preferred_element_type=jnp.float32) + b1_[...]
        h = h * jax.nn.sigmoid(h)
        radial.append(jnp.dot(h, w2_[...], preferred_element_type=jnp.float32) + b2_[...])
    r0, r1, r2 = radial
    s0 = sh0[...]
    s1 = sh1[...]
    s2 = sh2[...]
    rows_ = rows[...]

    st2 = jnp.concatenate(
        [s0 * r0[:, 0:32]]
        + [s1[:, m:m + 1] * r1[:, 0:32] for m in range(3)]
        + [s2[:, m:m + 1] * r2[:, 0:32] for m in range(5)], axis=1)
    st1 = jnp.concatenate(
        [s0 * r0[:, 32:64]]
        + [s1[:, m:m + 1] * r1[:, 32:64] for m in range(3)], axis=1)
    unc2 = jnp.dot(st2, m2[...], preferred_element_type=jnp.float32)
    unc1 = jnp.dot(st1, m1[...], preferred_element_type=jnp.float32)
    unc0 = u0[...] * s0 * r0[:, 64:128]

    outs = [unc0 * rows_[:, 0:64]]
    for i in range(2):
        acc = None
        for j in range(2):
            u = unc1[:, (2 * i + j) * 32:(2 * i + j + 1) * 32]
            t = jnp.concatenate([u, u], axis=1) * rows_[:, 64 + 64 * j:128 + 64 * j]
            acc = t if acc is None else acc + t
        outs.append(acc)
    for i in range(3):
        acc = None
        for j in range(3):
            u = unc2[:, (3 * i + j) * 32:(3 * i + j + 1) * 32]
            t = jnp.concatenate([u, u, u], axis=1) * rows_[:, 192 + 96 * j:288 + 96 * j]
            acc = t if acc is None else acc + t
        outs.append(acc)
    outs.append(jnp.zeros((rows_.shape[0], D - sum(COLS)), jnp.float32))
    out[...] = jnp.concatenate(outs, axis=1)


def _edge_compute(rb0, rb1, rb2, sh0, sh1, sh2, rows, weights):
    be = 1000
    grid = (N_EDGES // be,)
    rep = lambda i: (0, 0)
    blk = lambda i: (i, 0)
    w_specs = [pl.BlockSpec(w.shape, rep) for w in weights]
    return pl.pallas_call(
        _edge_body,
        grid=grid,
        in_specs=[
            pl.BlockSpec((be, 8), blk),
            pl.BlockSpec((be, 8), blk),
            pl.BlockSpec((be, 8), blk),
            pl.BlockSpec((be, 1), blk),
            pl.BlockSpec((be, 3), blk),
            pl.BlockSpec((be, 5), blk),
            pl.BlockSpec((be, D), blk),
        ] + w_specs,
        out_specs=pl.BlockSpec((be, D), blk),
        out_shape=jax.ShapeDtypeStruct((N_EDGES, D), jnp.float32),
    )(rb0, rb1, rb2, sh0, sh1, sh2, rows, *weights)


def _scatter_body(comb_hbm, ctr_hbm, zeros_hbm, out_hbm,
                  idx2d_v, rows_a, rows_b, acc_sh, sem_a, sem_b):
    c = lax.axis_index("c")
    s = lax.axis_index("s")
    bufs = (rows_a, rows_b)
    sems = (sem_a, sem_b)
    pltpu.sync_copy(ctr_hbm.at[s], idx2d_v)
    for g in range(4):
        @pl.when(c == g // 2)
        def _(g=g):
            col0 = g * GCOLS
            r0 = s * RPT
            pltpu.sync_copy(zeros_hbm.at[pl.ds(r0, RPT)],
                            acc_sh.at[pl.ds(r0, RPT)])
            plsc.subcore_barrier()

            def start(j, k):
                e0 = s * EPT + jnp.minimum(j, NSC - 1) * SCC
                pltpu.async_copy(
                    comb_hbm.at[pl.ds(e0, SCC), pl.ds(col0, GCOLS)],
                    bufs[k], sems[k])

            start(0, 0)
            start(1, 1)

            def body(jj, carry):
                for k in range(2):
                    j = jj * 2 + k
                    e0 = s * EPT + jnp.minimum(j, NSC - 1) * SCC
                    pltpu.make_async_copy(
                        comb_hbm.at[pl.ds(e0, SCC), pl.ds(col0, GCOLS)],
                        bufs[k], sems[k]).wait()
                    pltpu.sync_copy(bufs[k], acc_sh.at[idx2d_v.at[j]], add=True)
                    start(j + 2, k)
                return carry

            lax.fori_loop(0, NSC // 2, body, 0)
            for k in range(2):
                e0 = s * EPT
                pltpu.make_async_copy(
                    comb_hbm.at[pl.ds(e0, SCC), pl.ds(col0, GCOLS)],
                    bufs[k], sems[k]).wait()
            plsc.subcore_barrier()
            pltpu.sync_copy(
                acc_sh.at[pl.ds(r0, RPT)],
                out_hbm.at[pl.ds(r0, RPT), pl.ds(col0, GCOLS)])
            plsc.subcore_barrier()


def _sc_scatter(combined, centers3, zeros):
    return pl.kernel(
        _scatter_body,
        out_type=jax.ShapeDtypeStruct((N_POOL, D), jnp.float32),
        mesh=plsc.VectorSubcoreMesh(core_axis_name="c", subcore_axis_name="s"),
        scratch_types=[
            pltpu.VMEM((NSC, SCC), jnp.int32),
            pltpu.VMEM((SCC, GCOLS), jnp.float32),
            pltpu.VMEM((SCC, GCOLS), jnp.float32),
            pltpu.VMEM_SHARED((N_POOL, GCOLS), jnp.float32),
            pltpu.SemaphoreType.DMA,
            pltpu.SemaphoreType.DMA,
        ],
    )(combined, centers3, zeros)


def _final_body(pooled, f0, f1, f2, w0, w1, w2, ms, o0, o1, o2):
    p = pooled[...] * ms[...]
    o0[...] = f0[...] + jnp.dot(p[:, 0:64], w0[...], preferred_element_type=jnp.float32)
    o1[...] = f1[...] + jnp.dot(p[:, 64:192], w1[...], preferred_element_type=jnp.float32)
    o2[...] = f2[...] + jnp.dot(p[:, 192:480], w2[...], preferred_element_type=jnp.float32)


def _final(pooled, f0, f1, f2, w0, w1, w2, ms):
    bn = 1000
    grid = (N_ATOMS // bn,)
    rep = lambda i: (0, 0)
    blk = lambda i: (i, 0)
    return pl.pallas_call(
        _final_body,
        grid=grid,
        in_specs=[
            pl.BlockSpec((bn, D), blk),
            pl.BlockSpec((bn, 64), blk),
            pl.BlockSpec((bn, 128), blk),
            pl.BlockSpec((bn, 288), blk),
            pl.BlockSpec((64, 64), rep),
            pl.BlockSpec((128, 128), rep),
            pl.BlockSpec((288, 288), rep),
            pl.BlockSpec((1, 1), rep),
        ],
        out_specs=[
            pl.BlockSpec((bn, 64), blk),
            pl.BlockSpec((bn, 128), blk),
            pl.BlockSpec((bn, 288), blk),
        ],
        out_shape=[
            jax.ShapeDtypeStruct((N_ATOMS, 64), jnp.float32),
            jax.ShapeDtypeStruct((N_ATOMS, 128), jnp.float32),
            jax.ShapeDtypeStruct((N_ATOMS, 288), jnp.float32),
        ],
    )(pooled, f0, f1, f2, w0, w1, w2, ms)


def kernel(radial_basis_0, radial_basis_1, radial_basis_2,
           sh_0, sh_1, sh_2,
           features_0, features_1, features_2,
           U_0, U_1, U_2,
           mlp_W1_0, mlp_W1_1, mlp_W1_2,
           mlp_b1_0, mlp_b1_1, mlp_b1_2,
           mlp_W2_0, mlp_W2_1, mlp_W2_2,
           mlp_b2_0, mlp_b2_1, mlp_b2_2,
           lin_in_0, lin_in_1, lin_in_2,
           lin_out_0, lin_out_1, lin_out_2,
           rms_g_0, rms_g_1, rms_g_2,
           centers, neighbors, message_scaling, n_atoms):
    f32 = jnp.float32
    feats = (features_0, features_1, features_2)
    f_flat = [f.reshape(N_ATOMS, COLS[l]) for l, f in enumerate(feats)]
    g_flat = [g.reshape(1, COLS[l]) for l, g in
              enumerate((rms_g_0, rms_g_1, rms_g_2))]
    eye32 = jnp.eye(32, dtype=f32)
    lin_in_k = [lin_in_0,
                jnp.kron(jnp.eye(4, dtype=f32), lin_in_1),
                jnp.kron(jnp.eye(9, dtype=f32), lin_in_2)]
    lin_out_k = [lin_out_0,
                 jnp.kron(jnp.eye(4, dtype=f32), lin_out_1),
                 jnp.kron(jnp.eye(9, dtype=f32), lin_out_2)]
    m1 = jnp.kron(U_1.T, eye32) / math.sqrt(2.0)
    m2 = jnp.kron(U_2.T, eye32) / math.sqrt(3.0)
    u0 = U_0.reshape(1, 1)
    ms = (message_scaling
          + jnp.asarray(n_atoms - N_ATOMS, message_scaling.dtype)).reshape(1, 1)

    table = _node_prep(f_flat[0], f_flat[1], f_flat[2],
                       g_flat[0], g_flat[1], g_flat[2],
                       lin_in_k[0], lin_in_k[1], lin_in_k[2])
    rows = _sc_gather(table, neighbors)
    weights = [mlp_W1_0.reshape(8, 64), mlp_b1_0.reshape(1, 64),
               mlp_W1_1.reshape(8, 64), mlp_b1_1.reshape(1, 64),
               mlp_W1_2.reshape(8, 64), mlp_b1_2.reshape(1, 64),
               mlp_W2_0.reshape(64, 128), mlp_b2_0.reshape(1, 128),
               mlp_W2_1.reshape(64, 64), mlp_b2_1.reshape(1, 64),
               mlp_W2_2.reshape(64, 32), mlp_b2_2.reshape(1, 32),
               u0, m1, m2]
    combined = _edge_compute(radial_basis_0, radial_basis_1, radial_basis_2,
                             sh_0, sh_1, sh_2, rows, weights)
    zeros = jnp.zeros((N_POOL, GCOLS), f32)
    centers3 = centers.reshape(NS, NSC, SCC)
    pooled = _sc_scatter(combined, centers3, zeros)[:N_ATOMS]
    o0, o1, o2 = _final(pooled, f_flat[0], f_flat[1], f_flat[2],
                        lin_out_k[0], lin_out_k[1], lin_out_k[2], ms)
    return (o0.reshape(N_ATOMS, 1, 1, 64),
            o1.reshape(N_ATOMS, 2, 2, 32),
            o2.reshape(N_ATOMS, 3, 3, 32))


# speedup vs baseline: 74.7183x; 1.0950x over previous
import math
import jax
import jax.numpy as jnp
from jax import lax
from jax.experimental import pallas as pl
from jax.experimental.pallas import tpu as pltpu
from jax.experimental.pallas import tpu_sc as plsc

L_MAX = 2
K_MAX_L = (128, 64, 32)
WIDTHS = (64, 32, 32)
N_ATOMS = 10000
N_EDGES = 160000
EPS = 1e-6

D = 512
DP = D // 2
COLS = (64, 128, 288)
OFFS = (0, 64, 192)

NC, NS = 2, 16
NW = NC * NS

EPW = N_EDGES // NW
GC = 120
NGC = (EPW + GC - 1) // GC

EPT = N_EDGES // NS
SCC = 40
NSC = EPT // SCC
N_POOL = 10240
RPT = N_POOL // NS
GCOLS = 128


def _node_prep_body(f0, f1, f2, g0, g1, g2, w0, w1, w2, out):
    parts = []
    for f_ref, g_ref, w_ref in ((f0, g0, w0), (f1, g1, w1), (f2, g2, w2)):
        f = f_ref[...]
        ms = jnp.mean(f * f, axis=1, keepdims=True)
        f = f * lax.rsqrt(ms + EPS) * g_ref[...]
        parts.append(jnp.dot(f, w_ref[...], preferred_element_type=jnp.float32))
    parts.append(jnp.zeros((parts[0].shape[0], D - sum(COLS)), jnp.float32))
    y = jnp.concatenate(parts, axis=1)
    lo = lax.convert_element_type(
        lax.bitcast_convert_type(lax.convert_element_type(y[:, :DP], jnp.bfloat16),
                                 jnp.uint16), jnp.uint32)
    hi = lax.convert_element_type(
        lax.bitcast_convert_type(lax.convert_element_type(y[:, DP:], jnp.bfloat16),
                                 jnp.uint16), jnp.uint32)
    out[...] = lax.bitcast_convert_type(lo | (hi << 16), jnp.float32)


def _node_prep(f0, f1, f2, g0, g1, g2, w0, w1, w2):
    bn = 1000
    grid = (N_ATOMS // bn,)
    rep = lambda i: (0, 0)
    blk = lambda i: (i, 0)
    return pl.pallas_call(
        _node_prep_body,
        grid=grid,
        in_specs=[
            pl.BlockSpec((bn, 64), blk),
            pl.BlockSpec((bn, 128), blk),
            pl.BlockSpec((bn, 288), blk),
            pl.BlockSpec((1, 64), rep),
            pl.BlockSpec((1, 128), rep),
            pl.BlockSpec((1, 288), rep),
            pl.BlockSpec((64, 64), rep),
            pl.BlockSpec((128, 128), rep),
            pl.BlockSpec((288, 288), rep),
        ],
        out_specs=pl.BlockSpec((bn, DP), blk),
        out_shape=jax.ShapeDtypeStruct((N_ATOMS, DP), jnp.float32),
    )(f0, f1, f2, g0, g1, g2, w0, w1, w2)


def _gather_body(table_hbm, nbr_hbm, out_hbm, idx_v, rows_a, rows_b, sem_a, sem_b):
    c = lax.axis_index("c")
    s = lax.axis_index("s")
    wid = s * NC + c
    base = wid * EPW
    bufs = (rows_a, rows_b)
    sems = (sem_a, sem_b)
    pltpu.sync_copy(nbr_hbm.at[pl.ds(base, EPW)], idx_v)

    def start(j, k):
        e0 = jnp.minimum(j * GC, EPW - GC)
        pltpu.async_copy(table_hbm.at[idx_v.at[pl.ds(e0, GC)]], bufs[k], sems[k])

    start(0, 0)
    start(1, 1)

    def body(jj, carry):
        for k in range(2):
            j = jj * 2 + k
            e0 = jnp.minimum(j * GC, EPW - GC)
            pltpu.make_async_copy(table_hbm.at[idx_v.at[pl.ds(e0, GC)]],
                                  bufs[k], sems[k]).wait()
            pltpu.sync_copy(bufs[k], out_hbm.at[pl.ds(base + e0, GC)])
            start(j + 2, k)
        return carry

    lax.fori_loop(0, NGC // 2, body, 0)
    for k in range(2):
        pltpu.make_async_copy(table_hbm.at[idx_v.at[pl.ds(0, GC)]],
                              bufs[k], sems[k]).wait()


def _sc_gather(table, neighbors):
    return pl.kernel(
        _gather_body,
        out_type=jax.ShapeDtypeStruct((N_EDGES, DP), jnp.float32),
        mesh=plsc.VectorSubcoreMesh(core_axis_name="c", subcore_axis_name="s"),
        scratch_types=[
            pltpu.VMEM((EPW,), jnp.int32),
            pltpu.VMEM((GC, DP), jnp.float32),
            pltpu.VMEM((GC, DP), jnp.float32),
            pltpu.SemaphoreType.DMA,
            pltpu.SemaphoreType.DMA,
        ],
    )(table, neighbors)


def _edge_body(rb0, rb1, rb2, sh0, sh1, sh2, rows,
               w10, b10, w11, b11, w12, b12,
               w20, b20, w21, b21, w22, b22,
               u0, m1, m2, out):
    radial = []
    for rb, w1_, b1_, w2_, b2_ in ((rb0, w10, b10, w20, b20),
                                   (rb1, w11, b11, w21, b21),
                                   (rb2, w12, b12, w22, b22)):
        h = jnp.dot(rb[...], w1_[...], preferred_element_type=jnp.float32) + b1_[...]
        h = h * jax.nn.sigmoid(h)
        radial.append(jnp.dot(h, w2_[...], preferred_element_type=jnp.float32) + b2_[...])
    r0, r1, r2 = radial
    s0 = sh0[...]
    s1 = sh1[...]
    s2 = sh2[...]
    u = lax.bitcast_convert_type(rows[...], jnp.uint32)
    rows_ = jnp.concatenate(
        [lax.bitcast_convert_type(u << 16, jnp.float32),
         lax.bitcast_convert_type(u & jnp.uint32(0xFFFF0000), jnp.float32)],
        axis=1)

    st2 = jnp.concatenate(
        [s0 * r0[:, 0:32]]
        + [s1[:, m:m + 1] * r1[:, 0:32] for m in range(3)]
        + [s2[:, m:m + 1] * r2[:, 0:32] for m in range(5)], axis=1)
    st1 = jnp.concatenate(
        [s0 * r0[:, 32:64]]
        + [s1[:, m:m + 1] * r1[:, 32:64] for m in range(3)], axis=1)
    unc2 = jnp.dot(st2, m2[...], preferred_element_type=jnp.float32)
    unc1 = jnp.dot(st1, m1[...], preferred_element_type=jnp.float32)
    unc0 = u0[...] * s0 * r0[:, 64:128]

    outs = [unc0 * rows_[:, 0:64]]
    for i in range(2):
        acc = None
        for j in range(2):
            u = unc1[:, (2 * i + j) * 32:(2 * i + j + 1) * 32]
            t = jnp.concatenate([u, u], axis=1) * rows_[:, 64 + 64 * j:128 + 64 * j]
            acc = t if acc is None else acc + t
        outs.append(acc)
    for i in range(3):
        acc = None
        for j in range(3):
            u = unc2[:, (3 * i + j) * 32:(3 * i + j + 1) * 32]
            t = jnp.concatenate([u, u, u], axis=1) * rows_[:, 192 + 96 * j:288 + 96 * j]
            acc = t if acc is None else acc + t
        outs.append(acc)
    outs.append(jnp.zeros((rows_.shape[0], D - sum(COLS)), jnp.float32))
    out[...] = jnp.concatenate(outs, axis=1)


def _edge_compute(rb0, rb1, rb2, sh0, sh1, sh2, rows, weights):
    be = 1000
    grid = (N_EDGES // be,)
    rep = lambda i: (0, 0)
    blk = lambda i: (i, 0)
    w_specs = [pl.BlockSpec(w.shape, rep) for w in weights]
    return pl.pallas_call(
        _edge_body,
        grid=grid,
        in_specs=[
            pl.BlockSpec((be, 8), blk),
            pl.BlockSpec((be, 8), blk),
            pl.BlockSpec((be, 8), blk),
            pl.BlockSpec((be, 1), blk),
            pl.BlockSpec((be, 3), blk),
            pl.BlockSpec((be, 5), blk),
            pl.BlockSpec((be, DP), blk),
        ] + w_specs,
        out_specs=pl.BlockSpec((be, D), blk),
        out_shape=jax.ShapeDtypeStruct((N_EDGES, D), jnp.float32),
    )(rb0, rb1, rb2, sh0, sh1, sh2, rows, *weights)


def _scatter_body(comb_hbm, ctr_hbm, zeros_hbm, out_hbm,
                  idx2d_v, rows_a, rows_b, acc_sh, sem_a, sem_b):
    c = lax.axis_index("c")
    s = lax.axis_index("s")
    bufs = (rows_a, rows_b)
    sems = (sem_a, sem_b)
    pltpu.sync_copy(ctr_hbm.at[s], idx2d_v)
    for g in range(4):
        @pl.when(c == g // 2)
        def _(g=g):
            col0 = g * GCOLS
            r0 = s * RPT
            pltpu.sync_copy(zeros_hbm.at[pl.ds(r0, RPT)],
                            acc_sh.at[pl.ds(r0, RPT)])
            plsc.subcore_barrier()

            def start(j, k):
                e0 = s * EPT + jnp.minimum(j, NSC - 1) * SCC
                pltpu.async_copy(
                    comb_hbm.at[pl.ds(e0, SCC), pl.ds(col0, GCOLS)],
                    bufs[k], sems[k])

            start(0, 0)
            start(1, 1)

            def body(jj, carry):
                for k in range(2):
                    j = jj * 2 + k
                    e0 = s * EPT + jnp.minimum(j, NSC - 1) * SCC
                    pltpu.make_async_copy(
                        comb_hbm.at[pl.ds(e0, SCC), pl.ds(col0, GCOLS)],
                        bufs[k], sems[k]).wait()
                    pltpu.sync_copy(bufs[k], acc_sh.at[idx2d_v.at[j]], add=True)
                    start(j + 2, k)
                return carry

            lax.fori_loop(0, NSC // 2, body, 0)
            for k in range(2):
                e0 = s * EPT
                pltpu.make_async_copy(
                    comb_hbm.at[pl.ds(e0, SCC), pl.ds(col0, GCOLS)],
                    bufs[k], sems[k]).wait()
            plsc.subcore_barrier()
            pltpu.sync_copy(
                acc_sh.at[pl.ds(r0, RPT)],
                out_hbm.at[pl.ds(r0, RPT), pl.ds(col0, GCOLS)])
            plsc.subcore_barrier()


def _sc_scatter(combined, centers3, zeros):
    return pl.kernel(
        _scatter_body,
        out_type=jax.ShapeDtypeStruct((N_POOL, D), jnp.float32),
        mesh=plsc.VectorSubcoreMesh(core_axis_name="c", subcore_axis_name="s"),
        scratch_types=[
            pltpu.VMEM((NSC, SCC), jnp.int32),
            pltpu.VMEM((SCC, GCOLS), jnp.float32),
            pltpu.VMEM((SCC, GCOLS), jnp.float32),
            pltpu.VMEM_SHARED((N_POOL, GCOLS), jnp.float32),
            pltpu.SemaphoreType.DMA,
            pltpu.SemaphoreType.DMA,
        ],
    )(combined, centers3, zeros)


def _final_body(pooled, f0, f1, f2, w0, w1, w2, ms, o0, o1, o2):
    p = pooled[...] * ms[...]
    o0[...] = f0[...] + jnp.dot(p[:, 0:64], w0[...], preferred_element_type=jnp.float32)
    o1[...] = f1[...] + jnp.dot(p[:, 64:192], w1[...], preferred_element_type=jnp.float32)
    o2[...] = f2[...] + jnp.dot(p[:, 192:480], w2[...], preferred_element_type=jnp.float32)


def _final(pooled, f0, f1, f2, w0, w1, w2, ms):
    bn = 1000
    grid = (N_ATOMS // bn,)
    rep = lambda i: (0, 0)
    blk = lambda i: (i, 0)
    return pl.pallas_call(
        _final_body,
        grid=grid,
        in_specs=[
            pl.BlockSpec((bn, D), blk),
            pl.BlockSpec((bn, 64), blk),
            pl.BlockSpec((bn, 128), blk),
            pl.BlockSpec((bn, 288), blk),
            pl.BlockSpec((64, 64), rep),
            pl.BlockSpec((128, 128), rep),
            pl.BlockSpec((288, 288), rep),
            pl.BlockSpec((1, 1), rep),
        ],
        out_specs=[
            pl.BlockSpec((bn, 64), blk),
            pl.BlockSpec((bn, 128), blk),
            pl.BlockSpec((bn, 288), blk),
        ],
        out_shape=[
            jax.ShapeDtypeStruct((N_ATOMS, 64), jnp.float32),
            jax.ShapeDtypeStruct((N_ATOMS, 128), jnp.float32),
            jax.ShapeDtypeStruct((N_ATOMS, 288), jnp.float32),
        ],
    )(pooled, f0, f1, f2, w0, w1, w2, ms)


def kernel(radial_basis_0, radial_basis_1, radial_basis_2,
           sh_0, sh_1, sh_2,
           features_0, features_1, features_2,
           U_0, U_1, U_2,
           mlp_W1_0, mlp_W1_1, mlp_W1_2,
           mlp_b1_0, mlp_b1_1, mlp_b1_2,
           mlp_W2_0, mlp_W2_1, mlp_W2_2,
           mlp_b2_0, mlp_b2_1, mlp_b2_2,
           lin_in_0, lin_in_1, lin_in_2,
           lin_out_0, lin_out_1, lin_out_2,
           rms_g_0, rms_g_1, rms_g_2,
           centers, neighbors, message_scaling, n_atoms):
    f32 = jnp.float32
    feats = (features_0, features_1, features_2)
    f_flat = [f.reshape(N_ATOMS, COLS[l]) for l, f in enumerate(feats)]
    g_flat = [g.reshape(1, COLS[l]) for l, g in
              enumerate((rms_g_0, rms_g_1, rms_g_2))]
    eye32 = jnp.eye(32, dtype=f32)
    lin_in_k = [lin_in_0,
                jnp.kron(jnp.eye(4, dtype=f32), lin_in_1),
                jnp.kron(jnp.eye(9, dtype=f32), lin_in_2)]
    lin_out_k = [lin_out_0,
                 jnp.kron(jnp.eye(4, dtype=f32), lin_out_1),
                 jnp.kron(jnp.eye(9, dtype=f32), lin_out_2)]
    m1 = jnp.kron(U_1.T, eye32) / math.sqrt(2.0)
    m2 = jnp.kron(U_2.T, eye32) / math.sqrt(3.0)
    u0 = U_0.reshape(1, 1)
    ms = (message_scaling
          + jnp.asarray(n_atoms - N_ATOMS, message_scaling.dtype)).reshape(1, 1)

    table = _node_prep(f_flat[0], f_flat[1], f_flat[2],
                       g_flat[0], g_flat[1], g_flat[2],
                       lin_in_k[0], lin_in_k[1], lin_in_k[2])
    rows = _sc_gather(table, neighbors)
    weights = [mlp_W1_0.reshape(8, 64), mlp_b1_0.reshape(1, 64),
               mlp_W1_1.reshape(8, 64), mlp_b1_1.reshape(1, 64),
               mlp_W1_2.reshape(8, 64), mlp_b1_2.reshape(1, 64),
               mlp_W2_0.reshape(64, 128), mlp_b2_0.reshape(1, 128),
               mlp_W2_1.reshape(64, 64), mlp_b2_1.reshape(1, 64),
               mlp_W2_2.reshape(64, 32), mlp_b2_2.reshape(1, 32),
               u0, m1, m2]
    combined = _edge_compute(radial_basis_0, radial_basis_1, radial_basis_2,
                             sh_0, sh_1, sh_2, rows, weights)
    zeros = jnp.zeros((N_POOL, GCOLS), f32)
    centers3 = centers.reshape(NS, NSC, SCC)
    pooled = _sc_scatter(combined, centers3, zeros)[:N_ATOMS]
    o0, o1, o2 = _final(pooled, f_flat[0], f_flat[1], f_flat[2],
                        lin_out_k[0], lin_out_k[1], lin_out_k[2], ms)
    return (o0.reshape(N_ATOMS, 1, 1, 64),
            o1.reshape(N_ATOMS, 2, 2, 32),
            o2.reshape(N_ATOMS, 3, 3, 32))


# speedup vs baseline: 79.1372x; 1.0591x over previous
import math
import jax
import jax.numpy as jnp
from jax import lax
from jax.experimental import pallas as pl
from jax.experimental.pallas import tpu as pltpu
from jax.experimental.pallas import tpu_sc as plsc

L_MAX = 2
K_MAX_L = (128, 64, 32)
WIDTHS = (64, 32, 32)
N_ATOMS = 10000
N_EDGES = 160000
EPS = 1e-6

D = 512
DP = D // 2
COLS = (64, 128, 288)
OFFS = (0, 64, 192)

NC, NS = 2, 16
NW = NC * NS

E_HALF1 = 81920
E_HALF2 = N_EDGES - E_HALF1

GC = 120
SCC = 40
N_POOL = 10240
RPT = N_POOL // NS
GCOLS = 128


def _node_prep_body(f0, f1, f2, g0, g1, g2, w0, w1, w2, out):
    parts = []
    for f_ref, g_ref, w_ref in ((f0, g0, w0), (f1, g1, w1), (f2, g2, w2)):
        f = f_ref[...]
        ms = jnp.mean(f * f, axis=1, keepdims=True)
        f = f * lax.rsqrt(ms + EPS) * g_ref[...]
        parts.append(jnp.dot(f, w_ref[...], preferred_element_type=jnp.float32))
    parts.append(jnp.zeros((parts[0].shape[0], D - sum(COLS)), jnp.float32))
    y = jnp.concatenate(parts, axis=1)
    lo = lax.convert_element_type(
        lax.bitcast_convert_type(lax.convert_element_type(y[:, :DP], jnp.bfloat16),
                                 jnp.uint16), jnp.uint32)
    hi = lax.convert_element_type(
        lax.bitcast_convert_type(lax.convert_element_type(y[:, DP:], jnp.bfloat16),
                                 jnp.uint16), jnp.uint32)
    out[...] = lax.bitcast_convert_type(lo | (hi << 16), jnp.float32)


def _node_prep(f0, f1, f2, g0, g1, g2, w0, w1, w2):
    bn = 1000
    grid = (N_ATOMS // bn,)
    rep = lambda i: (0, 0)
    blk = lambda i: (i, 0)
    return pl.pallas_call(
        _node_prep_body,
        grid=grid,
        in_specs=[
            pl.BlockSpec((bn, 64), blk),
            pl.BlockSpec((bn, 128), blk),
            pl.BlockSpec((bn, 288), blk),
            pl.BlockSpec((1, 64), rep),
            pl.BlockSpec((1, 128), rep),
            pl.BlockSpec((1, 288), rep),
            pl.BlockSpec((64, 64), rep),
            pl.BlockSpec((128, 128), rep),
            pl.BlockSpec((288, 288), rep),
        ],
        out_specs=pl.BlockSpec((bn, DP), blk),
        out_shape=jax.ShapeDtypeStruct((N_ATOMS, DP), jnp.float32),
    )(f0, f1, f2, g0, g1, g2, w0, w1, w2)


def _sc_gather(table, neighbors, e_h):
    epw = e_h // NW
    ngc = (epw + GC - 1) // GC
    ngc += ngc % 2

    def body(table_hbm, nbr_hbm, out_hbm, idx_v, rows_a, rows_b, sem_a, sem_b):
        c = lax.axis_index("c")
        s = lax.axis_index("s")
        base = (s * NC + c) * epw
        bufs = (rows_a, rows_b)
        sems = (sem_a, sem_b)
        pltpu.sync_copy(nbr_hbm.at[pl.ds(base, epw)], idx_v)

        def start(j, k):
            e0 = jnp.minimum(j * GC, epw - GC)
            pltpu.async_copy(table_hbm.at[idx_v.at[pl.ds(e0, GC)]],
                             bufs[k], sems[k])

        start(0, 0)
        start(1, 1)

        def loop(jj, carry):
            for k in range(2):
                j = jj * 2 + k
                e0 = jnp.minimum(j * GC, epw - GC)
                pltpu.make_async_copy(table_hbm.at[idx_v.at[pl.ds(e0, GC)]],
                                      bufs[k], sems[k]).wait()
                pltpu.sync_copy(bufs[k], out_hbm.at[pl.ds(base + e0, GC)])
                start(j + 2, k)
            return carry

        lax.fori_loop(0, ngc // 2, loop, 0)
        for k in range(2):
            pltpu.make_async_copy(table_hbm.at[idx_v.at[pl.ds(0, GC)]],
                                  bufs[k], sems[k]).wait()

    return pl.kernel(
        body,
        out_type=jax.ShapeDtypeStruct((e_h, DP), jnp.float32),
        mesh=plsc.VectorSubcoreMesh(core_axis_name="c", subcore_axis_name="s"),
        scratch_types=[
            pltpu.VMEM((epw,), jnp.int32),
            pltpu.VMEM((GC, DP), jnp.float32),
            pltpu.VMEM((GC, DP), jnp.float32),
            pltpu.SemaphoreType.DMA,
            pltpu.SemaphoreType.DMA,
        ],
    )(table, neighbors)


def _edge_body(rb0, rb1, rb2, sh0, sh1, sh2, rows,
               w10, b10, w11, b11, w12, b12,
               w20, b20, w21, b21, w22, b22,
               u0, m1, m2, out):
    radial = []
    for rb, w1_, b1_, w2_, b2_ in ((rb0, w10, b10, w20, b20),
                                   (rb1, w11, b11, w21, b21),
                                   (rb2, w12, b12, w22, b22)):
        h = jnp.dot(rb[...], w1_[...], preferred_element_type=jnp.float32) + b1_[...]
        h = h * jax.nn.sigmoid(h)
        radial.append(jnp.dot(h, w2_[...], preferred_element_type=jnp.float32) + b2_[...])
    r0, r1, r2 = radial
    s0 = sh0[...]
    s1 = sh1[...]
    s2 = sh2[...]
    u = lax.bitcast_convert_type(rows[...], jnp.uint32)
    rows_ = jnp.concatenate(
        [lax.bitcast_convert_type(u << 16, jnp.float32),
         lax.bitcast_convert_type(u & jnp.uint32(0xFFFF0000), jnp.float32)],
        axis=1)

    st2 = jnp.concatenate(
        [s0 * r0[:, 0:32]]
        + [s1[:, m:m + 1] * r1[:, 0:32] for m in range(3)]
        + [s2[:, m:m + 1] * r2[:, 0:32] for m in range(5)], axis=1)
    st1 = jnp.concatenate(
        [s0 * r0[:, 32:64]]
        + [s1[:, m:m + 1] * r1[:, 32:64] for m in range(3)], axis=1)
    unc2 = jnp.dot(st2, m2[...], preferred_element_type=jnp.float32)
    unc1 = jnp.dot(st1, m1[...], preferred_element_type=jnp.float32)
    unc0 = u0[...] * s0 * r0[:, 64:128]

    outs = [unc0 * rows_[:, 0:64]]
    for i in range(2):
        acc = None
        for j in range(2):
            u = unc1[:, (2 * i + j) * 32:(2 * i + j + 1) * 32]
            t = jnp.concatenate([u, u], axis=1) * rows_[:, 64 + 64 * j:128 + 64 * j]
            acc = t if acc is None else acc + t
        outs.append(acc)
    for i in range(3):
        acc = None
        for j in range(3):
            u = unc2[:, (3 * i + j) * 32:(3 * i + j + 1) * 32]
            t = jnp.concatenate([u, u, u], axis=1) * rows_[:, 192 + 96 * j:288 + 96 * j]
            acc = t if acc is None else acc + t
        outs.append(acc)
    outs.append(jnp.zeros((rows_.shape[0], D - sum(COLS)), jnp.float32))
    out[...] = jnp.concatenate(outs, axis=1)


def _edge_compute(rb0, rb1, rb2, sh0, sh1, sh2, rows, weights, e_h):
    be = 640
    grid = (e_h // be,)
    rep = lambda i: (0, 0)
    blk = lambda i: (i, 0)
    w_specs = [pl.BlockSpec(w.shape, rep) for w in weights]
    return pl.pallas_call(
        _edge_body,
        grid=grid,
        in_specs=[
            pl.BlockSpec((be, 8), blk),
            pl.BlockSpec((be, 8), blk),
            pl.BlockSpec((be, 8), blk),
            pl.BlockSpec((be, 1), blk),
            pl.BlockSpec((be, 3), blk),
            pl.BlockSpec((be, 5), blk),
            pl.BlockSpec((be, DP), blk),
        ] + w_specs,
        out_specs=pl.BlockSpec((be, D), blk),
        out_shape=jax.ShapeDtypeStruct((e_h, D), jnp.float32),
    )(rb0, rb1, rb2, sh0, sh1, sh2, rows, *weights)


def _sc_scatter(combined, centers3, init, e_h):
    ept = e_h // NS
    nsc = ept // SCC

    def body(comb_hbm, ctr_hbm, init_hbm, out_hbm,
             idx2d_v, rows_a, rows_b, acc_sh, sem_a, sem_b):
        c = lax.axis_index("c")
        s = lax.axis_index("s")
        bufs = (rows_a, rows_b)
        sems = (sem_a, sem_b)
        pltpu.sync_copy(ctr_hbm.at[s], idx2d_v)
        for g in range(4):
            @pl.when(c == g // 2)
            def _(g=g):
                col0 = g * GCOLS
                r0 = s * RPT
                pltpu.sync_copy(
                    init_hbm.at[pl.ds(r0, RPT), pl.ds(col0, GCOLS)],
                    acc_sh.at[pl.ds(r0, RPT)])
                plsc.subcore_barrier()

                def start(j, k):
                    e0 = s * ept + jnp.minimum(j, nsc - 1) * SCC
                    pltpu.async_copy(
                        comb_hbm.at[pl.ds(e0, SCC), pl.ds(col0, GCOLS)],
                        bufs[k], sems[k])

                start(0, 0)
                start(1, 1)

                def loop(jj, carry):
                    for k in range(2):
                        j = jj * 2 + k
                        e0 = s * ept + jnp.minimum(j, nsc - 1) * SCC
                        pltpu.make_async_copy(
                            comb_hbm.at[pl.ds(e0, SCC), pl.ds(col0, GCOLS)],
                            bufs[k], sems[k]).wait()
                        pltpu.sync_copy(bufs[k], acc_sh.at[idx2d_v.at[j]],
                                        add=True)
                        start(j + 2, k)
                    return carry

                lax.fori_loop(0, nsc // 2, loop, 0)
                for k in range(2):
                    pltpu.make_async_copy(
                        comb_hbm.at[pl.ds(s * ept, SCC), pl.ds(col0, GCOLS)],
                        bufs[k], sems[k]).wait()
                plsc.subcore_barrier()
                pltpu.sync_copy(
                    acc_sh.at[pl.ds(r0, RPT)],
                    out_hbm.at[pl.ds(r0, RPT), pl.ds(col0, GCOLS)])
                plsc.subcore_barrier()

    return pl.kernel(
        body,
        out_type=jax.ShapeDtypeStruct((N_POOL, D), jnp.float32),
        mesh=plsc.VectorSubcoreMesh(core_axis_name="c", subcore_axis_name="s"),
        scratch_types=[
            pltpu.VMEM((nsc, SCC), jnp.int32),
            pltpu.VMEM((SCC, GCOLS), jnp.float32),
            pltpu.VMEM((SCC, GCOLS), jnp.float32),
            pltpu.VMEM_SHARED((N_POOL, GCOLS), jnp.float32),
            pltpu.SemaphoreType.DMA,
            pltpu.SemaphoreType.DMA,
        ],
    )(combined, centers3, init)


def _final_body(pooled, f0, f1, f2, w0, w1, w2, ms, o0, o1, o2):
    p = pooled[...] * ms[...]
    o0[...] = f0[...] + jnp.dot(p[:, 0:64], w0[...], preferred_element_type=jnp.float32)
    o1[...] = f1[...] + jnp.dot(p[:, 64:192], w1[...], preferred_element_type=jnp.float32)
    o2[...] = f2[...] + jnp.dot(p[:, 192:480], w2[...], preferred_element_type=jnp.float32)


def _final(pooled, f0, f1, f2, w0, w1, w2, ms):
    bn = 1000
    grid = (N_ATOMS // bn,)
    rep = lambda i: (0, 0)
    blk = lambda i: (i, 0)
    return pl.pallas_call(
        _final_body,
        grid=grid,
        in_specs=[
            pl.BlockSpec((bn, D), blk),
            pl.BlockSpec((bn, 64), blk),
            pl.BlockSpec((bn, 128), blk),
            pl.BlockSpec((bn, 288), blk),
            pl.BlockSpec((64, 64), rep),
            pl.BlockSpec((128, 128), rep),
            pl.BlockSpec((288, 288), rep),
            pl.BlockSpec((1, 1), rep),
        ],
        out_specs=[
            pl.BlockSpec((bn, 64), blk),
            pl.BlockSpec((bn, 128), blk),
            pl.BlockSpec((bn, 288), blk),
        ],
        out_shape=[
            jax.ShapeDtypeStruct((N_ATOMS, 64), jnp.float32),
            jax.ShapeDtypeStruct((N_ATOMS, 128), jnp.float32),
            jax.ShapeDtypeStruct((N_ATOMS, 288), jnp.float32),
        ],
    )(pooled, f0, f1, f2, w0, w1, w2, ms)


def kernel(radial_basis_0, radial_basis_1, radial_basis_2,
           sh_0, sh_1, sh_2,
           features_0, features_1, features_2,
           U_0, U_1, U_2,
           mlp_W1_0, mlp_W1_1, mlp_W1_2,
           mlp_b1_0, mlp_b1_1, mlp_b1_2,
           mlp_W2_0, mlp_W2_1, mlp_W2_2,
           mlp_b2_0, mlp_b2_1, mlp_b2_2,
           lin_in_0, lin_in_1, lin_in_2,
           lin_out_0, lin_out_1, lin_out_2,
           rms_g_0, rms_g_1, rms_g_2,
           centers, neighbors, message_scaling, n_atoms):
    f32 = jnp.float32
    feats = (features_0, features_1, features_2)
    f_flat = [f.reshape(N_ATOMS, COLS[l]) for l, f in enumerate(feats)]
    g_flat = [g.reshape(1, COLS[l]) for l, g in
              enumerate((rms_g_0, rms_g_1, rms_g_2))]
    eye32 = jnp.eye(32, dtype=f32)
    lin_in_k = [lin_in_0,
                jnp.kron(jnp.eye(4, dtype=f32), lin_in_1),
                jnp.kron(jnp.eye(9, dtype=f32), lin_in_2)]
    lin_out_k = [lin_out_0,
                 jnp.kron(jnp.eye(4, dtype=f32), lin_out_1),
                 jnp.kron(jnp.eye(9, dtype=f32), lin_out_2)]
    m1 = jnp.kron(U_1.T, eye32) / math.sqrt(2.0)
    m2 = jnp.kron(U_2.T, eye32) / math.sqrt(3.0)
    u0 = U_0.reshape(1, 1)
    ms = (message_scaling
          + jnp.asarray(n_atoms - N_ATOMS, message_scaling.dtype)).reshape(1, 1)

    table = _node_prep(f_flat[0], f_flat[1], f_flat[2],
                       g_flat[0], g_flat[1], g_flat[2],
                       lin_in_k[0], lin_in_k[1], lin_in_k[2])
    weights = [mlp_W1_0.reshape(8, 64), mlp_b1_0.reshape(1, 64),
               mlp_W1_1.reshape(8, 64), mlp_b1_1.reshape(1, 64),
               mlp_W1_2.reshape(8, 64), mlp_b1_2.reshape(1, 64),
               mlp_W2_0.reshape(64, 128), mlp_b2_0.reshape(1, 128),
               mlp_W2_1.reshape(64, 64), mlp_b2_1.reshape(1, 64),
               mlp_W2_2.reshape(64, 32), mlp_b2_2.reshape(1, 32),
               u0, m1, m2]
    pooled = jnp.zeros((N_POOL, D), f32)
    for b, e_h in ((0, E_HALF1), (E_HALF1, E_HALF2)):
        rows = _sc_gather(table, lax.slice(neighbors, (b,), (b + e_h,)), e_h)
        combined = _edge_compute(
            lax.slice(radial_basis_0, (b, 0), (b + e_h, 8)),
            lax.slice(radial_basis_1, (b, 0), (b + e_h, 8)),
            lax.slice(radial_basis_2, (b, 0), (b + e_h, 8)),
            lax.slice(sh_0, (b, 0), (b + e_h, 1)),
            lax.slice(sh_1, (b, 0), (b + e_h, 3)),
            lax.slice(sh_2, (b, 0), (b + e_h, 5)),
            rows, weights, e_h)
        centers3 = lax.slice(centers, (b,), (b + e_h,)).reshape(
            NS, e_h // NS // SCC, SCC)
        pooled = _sc_scatter(combined, centers3, pooled, e_h)
    pooled = pooled[:N_ATOMS]
    o0, o1, o2 = _final(pooled, f_flat[0], f_flat[1], f_flat[2],
                        lin_out_k[0], lin_out_k[1], lin_out_k[2], ms)
    return (o0.reshape(N_ATOMS, 1, 1, 64),
            o1.reshape(N_ATOMS, 2, 2, 32),
            o2.reshape(N_ATOMS, 3, 3, 32))


# speedup vs baseline: 80.8005x; 1.0210x over previous
import math
import jax
import jax.numpy as jnp
from jax import lax
from jax.experimental import pallas as pl
from jax.experimental.pallas import tpu as pltpu
from jax.experimental.pallas import tpu_sc as plsc

L_MAX = 2
K_MAX_L = (128, 64, 32)
WIDTHS = (64, 32, 32)
N_ATOMS = 10000
N_EDGES = 160000
EPS = 1e-6

D = 512
DP = D // 2
COLS = (64, 128, 288)
OFFS = (0, 64, 192)

NC, NS = 2, 16
NW = NC * NS

E_HALF1 = 81920
E_HALF2 = N_EDGES - E_HALF1

GC = 120
SCC = 40
N_POOL = 10240
RPT = N_POOL // NS
GCOLS = 128


def _node_prep_body(f0, f1, f2, g0, g1, g2, w0, w1, w2, out):
    parts = []
    for f_ref, g_ref, w_ref in ((f0, g0, w0), (f1, g1, w1), (f2, g2, w2)):
        f = f_ref[...]
        ms = jnp.mean(f * f, axis=1, keepdims=True)
        f = f * lax.rsqrt(ms + EPS) * g_ref[...]
        parts.append(jnp.dot(f, w_ref[...], preferred_element_type=jnp.float32))
    parts.append(jnp.zeros((parts[0].shape[0], D - sum(COLS)), jnp.float32))
    y = jnp.concatenate(parts, axis=1)
    lo = lax.convert_element_type(
        lax.bitcast_convert_type(lax.convert_element_type(y[:, :DP], jnp.bfloat16),
                                 jnp.uint16), jnp.uint32)
    hi = lax.convert_element_type(
        lax.bitcast_convert_type(lax.convert_element_type(y[:, DP:], jnp.bfloat16),
                                 jnp.uint16), jnp.uint32)
    out[...] = lax.bitcast_convert_type(lo | (hi << 16), jnp.float32)


def _node_prep(f0, f1, f2, g0, g1, g2, w0, w1, w2):
    bn = 1000
    grid = (N_ATOMS // bn,)
    rep = lambda i: (0, 0)
    blk = lambda i: (i, 0)
    return pl.pallas_call(
        _node_prep_body,
        grid=grid,
        in_specs=[
            pl.BlockSpec((bn, 64), blk),
            pl.BlockSpec((bn, 128), blk),
            pl.BlockSpec((bn, 288), blk),
            pl.BlockSpec((1, 64), rep),
            pl.BlockSpec((1, 128), rep),
            pl.BlockSpec((1, 288), rep),
            pl.BlockSpec((64, 64), rep),
            pl.BlockSpec((128, 128), rep),
            pl.BlockSpec((288, 288), rep),
        ],
        out_specs=pl.BlockSpec((bn, DP), blk),
        out_shape=jax.ShapeDtypeStruct((N_ATOMS, DP), jnp.float32),
    )(f0, f1, f2, g0, g1, g2, w0, w1, w2)


def _sc_gather(table, neighbors, e_h):
    epw = e_h // NW
    ngc = (epw + GC - 1) // GC
    ngc += ngc % 2

    def body(table_hbm, nbr_hbm, out_hbm, idx_v, rows_a, rows_b, sem_a, sem_b):
        c = lax.axis_index("c")
        s = lax.axis_index("s")
        base = (s * NC + c) * epw
        bufs = (rows_a, rows_b)
        sems = (sem_a, sem_b)
        pltpu.sync_copy(nbr_hbm.at[pl.ds(base, epw)], idx_v)

        def start(j, k):
            e0 = jnp.minimum(j * GC, epw - GC)
            pltpu.async_copy(table_hbm.at[idx_v.at[pl.ds(e0, GC)]],
                             bufs[k], sems[k])

        start(0, 0)
        start(1, 1)

        def loop(jj, carry):
            for k in range(2):
                j = jj * 2 + k
                e0 = jnp.minimum(j * GC, epw - GC)
                pltpu.make_async_copy(table_hbm.at[idx_v.at[pl.ds(e0, GC)]],
                                      bufs[k], sems[k]).wait()
                pltpu.sync_copy(bufs[k], out_hbm.at[pl.ds(base + e0, GC)])
                start(j + 2, k)
            return carry

        lax.fori_loop(0, ngc // 2, loop, 0)
        for k in range(2):
            pltpu.make_async_copy(table_hbm.at[idx_v.at[pl.ds(0, GC)]],
                                  bufs[k], sems[k]).wait()

    return pl.kernel(
        body,
        out_type=jax.ShapeDtypeStruct((e_h, DP), jnp.float32),
        mesh=plsc.VectorSubcoreMesh(core_axis_name="c", subcore_axis_name="s"),
        scratch_types=[
            pltpu.VMEM((epw,), jnp.int32),
            pltpu.VMEM((GC, DP), jnp.float32),
            pltpu.VMEM((GC, DP), jnp.float32),
            pltpu.SemaphoreType.DMA,
            pltpu.SemaphoreType.DMA,
        ],
    )(table, neighbors)


def _edge_body(rb0, rb1, rb2, sh0, sh1, sh2, rows,
               w10, b10, w11, b11, w12, b12,
               w20, b20, w21, b21, w22, b22,
               u0, m1, m2, out):
    radial = []
    for rb, w1_, b1_, w2_, b2_ in ((rb0, w10, b10, w20, b20),
                                   (rb1, w11, b11, w21, b21),
                                   (rb2, w12, b12, w22, b22)):
        h = jnp.dot(rb[...], w1_[...], preferred_element_type=jnp.float32) + b1_[...]
        h = h * jax.nn.sigmoid(h)
        radial.append(jnp.dot(h, w2_[...], preferred_element_type=jnp.float32) + b2_[...])
    r0, r1, r2 = radial
    s0 = sh0[...]
    s1 = sh1[...]
    s2 = sh2[...]
    u = lax.bitcast_convert_type(rows[...], jnp.uint32)
    rows_ = jnp.concatenate(
        [lax.bitcast_convert_type(u << 16, jnp.float32),
         lax.bitcast_convert_type(u & jnp.uint32(0xFFFF0000), jnp.float32)],
        axis=1)

    st2 = jnp.concatenate(
        [s0 * r0[:, 0:32]]
        + [s1[:, m:m + 1] * r1[:, 0:32] for m in range(3)]
        + [s2[:, m:m + 1] * r2[:, 0:32] for m in range(5)], axis=1)
    st1 = jnp.concatenate(
        [s0 * r0[:, 32:64]]
        + [s1[:, m:m + 1] * r1[:, 32:64] for m in range(3)], axis=1)
    unc2 = jnp.dot(st2, m2[...], preferred_element_type=jnp.float32)
    unc1 = jnp.dot(st1, m1[...], preferred_element_type=jnp.float32)
    unc0 = u0[...] * s0 * r0[:, 64:128]

    outs = [unc0 * rows_[:, 0:64]]
    for i in range(2):
        acc = None
        for j in range(2):
            u = unc1[:, (2 * i + j) * 32:(2 * i + j + 1) * 32]
            t = jnp.concatenate([u, u], axis=1) * rows_[:, 64 + 64 * j:128 + 64 * j]
            acc = t if acc is None else acc + t
        outs.append(acc)
    for i in range(3):
        acc = None
        for j in range(3):
            u = unc2[:, (3 * i + j) * 32:(3 * i + j + 1) * 32]
            t = jnp.concatenate([u, u, u], axis=1) * rows_[:, 192 + 96 * j:288 + 96 * j]
            acc = t if acc is None else acc + t
        outs.append(acc)
    outs.append(jnp.zeros((rows_.shape[0], D - sum(COLS)), jnp.float32))
    out[...] = jnp.concatenate(outs, axis=1)


def _edge_compute(rb0, rb1, rb2, sh0, sh1, sh2, rows, weights, e_h):
    be = 1280
    grid = (e_h // be,)
    rep = lambda i: (0, 0)
    blk = lambda i: (i, 0)
    w_specs = [pl.BlockSpec(w.shape, rep) for w in weights]
    return pl.pallas_call(
        _edge_body,
        grid=grid,
        in_specs=[
            pl.BlockSpec((be, 8), blk),
            pl.BlockSpec((be, 8), blk),
            pl.BlockSpec((be, 8), blk),
            pl.BlockSpec((be, 1), blk),
            pl.BlockSpec((be, 3), blk),
            pl.BlockSpec((be, 5), blk),
            pl.BlockSpec((be, DP), blk),
        ] + w_specs,
        out_specs=pl.BlockSpec((be, D), blk),
        out_shape=jax.ShapeDtypeStruct((e_h, D), jnp.float32),
    )(rb0, rb1, rb2, sh0, sh1, sh2, rows, *weights)


def _sc_scatter(combined, centers3, init, e_h):
    ept = e_h // NS
    nsc = ept // SCC

    def body(comb_hbm, ctr_hbm, init_hbm, out_hbm,
             idx2d_v, rows_a, rows_b, acc_sh, sem_a, sem_b):
        c = lax.axis_index("c")
        s = lax.axis_index("s")
        bufs = (rows_a, rows_b)
        sems = (sem_a, sem_b)
        pltpu.sync_copy(ctr_hbm.at[s], idx2d_v)
        for g in range(4):
            @pl.when(c == g // 2)
            def _(g=g):
                col0 = g * GCOLS
                r0 = s * RPT
                pltpu.sync_copy(
                    init_hbm.at[pl.ds(r0, RPT), pl.ds(col0, GCOLS)],
                    acc_sh.at[pl.ds(r0, RPT)])
                plsc.subcore_barrier()

                def start(j, k):
                    e0 = s * ept + jnp.minimum(j, nsc - 1) * SCC
                    pltpu.async_copy(
                        comb_hbm.at[pl.ds(e0, SCC), pl.ds(col0, GCOLS)],
                        bufs[k], sems[k])

                start(0, 0)
                start(1, 1)

                def loop(jj, carry):
                    for k in range(2):
                        j = jj * 2 + k
                        e0 = s * ept + jnp.minimum(j, nsc - 1) * SCC
                        pltpu.make_async_copy(
                            comb_hbm.at[pl.ds(e0, SCC), pl.ds(col0, GCOLS)],
                            bufs[k], sems[k]).wait()
                        pltpu.sync_copy(bufs[k], acc_sh.at[idx2d_v.at[j]],
                                        add=True)
                        start(j + 2, k)
                    return carry

                lax.fori_loop(0, nsc // 2, loop, 0)
                for k in range(2):
                    pltpu.make_async_copy(
                        comb_hbm.at[pl.ds(s * ept, SCC), pl.ds(col0, GCOLS)],
                        bufs[k], sems[k]).wait()
                plsc.subcore_barrier()
                pltpu.sync_copy(
                    acc_sh.at[pl.ds(r0, RPT)],
                    out_hbm.at[pl.ds(r0, RPT), pl.ds(col0, GCOLS)])
                plsc.subcore_barrier()

    return pl.kernel(
        body,
        out_type=jax.ShapeDtypeStruct((N_POOL, D), jnp.float32),
        mesh=plsc.VectorSubcoreMesh(core_axis_name="c", subcore_axis_name="s"),
        scratch_types=[
            pltpu.VMEM((nsc, SCC), jnp.int32),
            pltpu.VMEM((SCC, GCOLS), jnp.float32),
            pltpu.VMEM((SCC, GCOLS), jnp.float32),
            pltpu.VMEM_SHARED((N_POOL, GCOLS), jnp.float32),
            pltpu.SemaphoreType.DMA,
            pltpu.SemaphoreType.DMA,
        ],
    )(combined, centers3, init)


def _final_body(pooled, f0, f1, f2, w0, w1, w2, ms, o0, o1, o2):
    p = pooled[...] * ms[...]
    o0[...] = f0[...] + jnp.dot(p[:, 0:64], w0[...], preferred_element_type=jnp.float32)
    o1[...] = f1[...] + jnp.dot(p[:, 64:192], w1[...], preferred_element_type=jnp.float32)
    o2[...] = f2[...] + jnp.dot(p[:, 192:480], w2[...], preferred_element_type=jnp.float32)


def _final(pooled, f0, f1, f2, w0, w1, w2, ms):
    bn = 1000
    grid = (N_ATOMS // bn,)
    rep = lambda i: (0, 0)
    blk = lambda i: (i, 0)
    return pl.pallas_call(
        _final_body,
        grid=grid,
        in_specs=[
            pl.BlockSpec((bn, D), blk),
            pl.BlockSpec((bn, 64), blk),
            pl.BlockSpec((bn, 128), blk),
            pl.BlockSpec((bn, 288), blk),
            pl.BlockSpec((64, 64), rep),
            pl.BlockSpec((128, 128), rep),
            pl.BlockSpec((288, 288), rep),
            pl.BlockSpec((1, 1), rep),
        ],
        out_specs=[
            pl.BlockSpec((bn, 64), blk),
            pl.BlockSpec((bn, 128), blk),
            pl.BlockSpec((bn, 288), blk),
        ],
        out_shape=[
            jax.ShapeDtypeStruct((N_ATOMS, 64), jnp.float32),
            jax.ShapeDtypeStruct((N_ATOMS, 128), jnp.float32),
            jax.ShapeDtypeStruct((N_ATOMS, 288), jnp.float32),
        ],
    )(pooled, f0, f1, f2, w0, w1, w2, ms)


def kernel(radial_basis_0, radial_basis_1, radial_basis_2,
           sh_0, sh_1, sh_2,
           features_0, features_1, features_2,
           U_0, U_1, U_2,
           mlp_W1_0, mlp_W1_1, mlp_W1_2,
           mlp_b1_0, mlp_b1_1, mlp_b1_2,
           mlp_W2_0, mlp_W2_1, mlp_W2_2,
           mlp_b2_0, mlp_b2_1, mlp_b2_2,
           lin_in_0, lin_in_1, lin_in_2,
           lin_out_0, lin_out_1, lin_out_2,
           rms_g_0, rms_g_1, rms_g_2,
           centers, neighbors, message_scaling, n_atoms):
    f32 = jnp.float32
    feats = (features_0, features_1, features_2)
    f_flat = [f.reshape(N_ATOMS, COLS[l]) for l, f in enumerate(feats)]
    g_flat = [g.reshape(1, COLS[l]) for l, g in
              enumerate((rms_g_0, rms_g_1, rms_g_2))]
    eye32 = jnp.eye(32, dtype=f32)
    lin_in_k = [lin_in_0,
                jnp.kron(jnp.eye(4, dtype=f32), lin_in_1),
                jnp.kron(jnp.eye(9, dtype=f32), lin_in_2)]
    lin_out_k = [lin_out_0,
                 jnp.kron(jnp.eye(4, dtype=f32), lin_out_1),
                 jnp.kron(jnp.eye(9, dtype=f32), lin_out_2)]
    m1 = jnp.kron(U_1.T, eye32) / math.sqrt(2.0)
    m2 = jnp.kron(U_2.T, eye32) / math.sqrt(3.0)
    u0 = U_0.reshape(1, 1)
    ms = (message_scaling
          + jnp.asarray(n_atoms - N_ATOMS, message_scaling.dtype)).reshape(1, 1)

    table = _node_prep(f_flat[0], f_flat[1], f_flat[2],
                       g_flat[0], g_flat[1], g_flat[2],
                       lin_in_k[0], lin_in_k[1], lin_in_k[2])
    weights = [mlp_W1_0.reshape(8, 64), mlp_b1_0.reshape(1, 64),
               mlp_W1_1.reshape(8, 64), mlp_b1_1.reshape(1, 64),
               mlp_W1_2.reshape(8, 64), mlp_b1_2.reshape(1, 64),
               mlp_W2_0.reshape(64, 128), mlp_b2_0.reshape(1, 128),
               mlp_W2_1.reshape(64, 64), mlp_b2_1.reshape(1, 64),
               mlp_W2_2.reshape(64, 32), mlp_b2_2.reshape(1, 32),
               u0, m1, m2]
    pooled = jnp.zeros((N_POOL, D), f32)
    for b, e_h in ((0, E_HALF1), (E_HALF1, E_HALF2)):
        rows = _sc_gather(table, lax.slice(neighbors, (b,), (b + e_h,)), e_h)
        combined = _edge_compute(
            lax.slice(radial_basis_0, (b, 0), (b + e_h, 8)),
            lax.slice(radial_basis_1, (b, 0), (b + e_h, 8)),
            lax.slice(radial_basis_2, (b, 0), (b + e_h, 8)),
            lax.slice(sh_0, (b, 0), (b + e_h, 1)),
            lax.slice(sh_1, (b, 0), (b + e_h, 3)),
            lax.slice(sh_2, (b, 0), (b + e_h, 5)),
            rows, weights, e_h)
        centers3 = lax.slice(centers, (b,), (b + e_h,)).reshape(
            NS, e_h // NS // SCC, SCC)
        pooled = _sc_scatter(combined, centers3, pooled, e_h)
    pooled = pooled[:N_ATOMS]
    o0, o1, o2 = _final(pooled, f_flat[0], f_flat[1], f_flat[2],
                        lin_out_k[0], lin_out_k[1], lin_out_k[2], ms)
    return (o0.reshape(N_ATOMS, 1, 1, 64),
            o1.reshape(N_ATOMS, 2, 2, 32),
            o2.reshape(N_ATOMS, 3, 3, 32))
